# baseline (speedup 1.0000x reference)
import functools

import jax
import jax.numpy as jnp
from jax.experimental import pallas as pl
from jax.experimental.pallas import tpu as pltpu

_B, _S = 128, 64
_D = 64
_P = 512
_K = 8192
_N = _B * _S
_EPS = 0.1

_R = 2048
_KT = 1024


def _bound_body(o_ref, w_ref, b_ref, m_out_ref):
    ones = jnp.ones((_P, 1), jnp.bfloat16)
    o = o_ref[...]
    o2 = jax.lax.dot_general(o * o, ones, (((1,), (0,)), ((), ())),
                             preferred_element_type=jnp.float32)
    w = w_ref[...]
    w2 = jax.lax.dot_general(w * w, ones, (((1,), (0,)), ((), ())),
                             preferred_element_type=jnp.float32)
    omax = jnp.max(o2, keepdims=True)
    wmax = jnp.max(w2, keepdims=True)
    bmax = jnp.max(jnp.abs(b_ref[...]), keepdims=True)
    m_out_ref[...] = jnp.sqrt(omax * wmax) * 1.05 + bmax + 0.05


def _fused_body(m_in_ref, x_ref, c_ref, o_ref, w_ref, b_ref, mask_ref,
                out_ref, accn_ref, accd_ref, wsum_ref, bsum_ref, ca_ref):
    i = pl.program_id(0)
    ni = pl.num_programs(0)

    @pl.when(i == 0)
    def _init():
        accn_ref[...] = jnp.zeros_like(accn_ref)
        accd_ref[...] = jnp.zeros_like(accd_ref)
        wsum_ref[...] = jnp.sum(w_ref[...].astype(jnp.float32), axis=0,
                                keepdims=True)
        bsum_ref[...] = jnp.sum(b_ref[...], keepdims=True)
        cf = c_ref[...]
        c2f = jnp.sum(cf * cf, axis=0, keepdims=True)
        ca_ref[...] = jnp.concatenate(
            [cf.astype(jnp.bfloat16), (-0.5 * c2f).astype(jnp.bfloat16)],
            axis=0)

    mb = m_in_ref[0, 0].astype(jnp.bfloat16)
    xa = x_ref[...]
    o = o_ref[...]

    ones = jnp.ones((_KT, 1), jnp.bfloat16)
    _rowsum = lambda t: jax.lax.dot_general(
        t, ones, (((1,), (0,)), ((), ())), preferred_element_type=jnp.float32)

    esum = jnp.zeros((_R, _KT), jnp.bfloat16)
    best = jnp.full((_R, 1), -jnp.inf, jnp.float32)
    bp = jnp.zeros((_R, 1), jnp.float32)
    for jj in range(_K // _KT):
        ksl = slice(jj * _KT, (jj + 1) * _KT)
        ca = ca_ref[:, ksl]
        score = jnp.dot(xa, ca, preferred_element_type=jnp.float32)

        w = w_ref[ksl, :]
        pred32 = jax.lax.dot_general(o, w, (((1,), (1,)), ((), ())),
                                     preferred_element_type=jnp.float32)
        pred = (pred32 + b_ref[:, ksl]).astype(jnp.bfloat16)

        e = jnp.exp(pred - mb)
        esum = esum + e

        tb = jnp.max(score, axis=1, keepdims=True)
        pa = _rowsum(jnp.where(score == tb, pred, 0.0))
        upd = tb > best
        bp = jnp.where(upd, pa, bp)
        best = jnp.maximum(best, tb)

    s = _rowsum(esum)
    lse = mb.astype(jnp.float32) + jnp.log(s)
    nll = lse - bp
    ow = o.astype(jnp.float32) * wsum_ref[...]
    ps = jax.lax.dot_general(
        ow, jnp.ones((_P, 1), jnp.float32), (((1,), (0,)), ((), ())),
        preferred_element_type=jnp.float32) + bsum_ref[...]
    smooth = lse - ps * (1.0 / _K)
    elem = (1.0 - _EPS) * nll + _EPS * smooth
    mk = mask_ref[...]
    accn_ref[...] = accn_ref[...] + jnp.sum(mk * elem, keepdims=True)
    accd_ref[...] = accd_ref[...] + jnp.sum(mk, keepdims=True)

    @pl.when(i == ni - 1)
    def _emit():
        out_ref[...] = accn_ref[...] / accd_ref[...]


@functools.partial(jax.jit, static_argnames=("interpret",))
def _fused_loss(x, mask_f, o, centroids, head_w, head_b, interpret=False):
    m_bound = pl.pallas_call(
        _bound_body,
        out_shape=jax.ShapeDtypeStruct((1, 1), jnp.float32),
        interpret=interpret,
    )(o, head_w, head_b)

    grid = (_N // _R,)
    out = pl.pallas_call(
        _fused_body,
        grid=grid,
        in_specs=[
            pl.BlockSpec((1, 1), lambda i: (0, 0)),
            pl.BlockSpec((_R, _D + 1), lambda i: (i, 0)),
            pl.BlockSpec((_D, _K), lambda i: (0, 0)),
            pl.BlockSpec((_R, _P), lambda i: (i, 0)),
            pl.BlockSpec((_K, _P), lambda i: (0, 0)),
            pl.BlockSpec((1, _K), lambda i: (0, 0)),
            pl.BlockSpec((_R, 1), lambda i: (i, 0)),
        ],
        out_specs=pl.BlockSpec((1, 1), lambda i: (0, 0)),
        out_shape=jax.ShapeDtypeStruct((1, 1), jnp.float32),
        scratch_shapes=[
            pltpu.VMEM((1, 1), jnp.float32),
            pltpu.VMEM((1, 1), jnp.float32),
            pltpu.VMEM((1, _P), jnp.float32),
            pltpu.VMEM((1, 1), jnp.float32),
            pltpu.VMEM((_D + 1, _K), jnp.bfloat16),
        ],
        compiler_params=pltpu.CompilerParams(
            dimension_semantics=("arbitrary",),
        ),
        interpret=interpret,
    )(m_bound, x, centroids, o, head_w, head_b, mask_f)
    return out[0, 0]


def kernel(csts, null_mask, outputs, centroids, head_w, head_b):
    x = jnp.concatenate(
        [csts.reshape(_N, _D), jnp.ones((_N, 1), jnp.float32)],
        axis=1).astype(jnp.bfloat16)
    o = outputs.reshape(_N, _P).astype(jnp.bfloat16)
    head_w = head_w.astype(jnp.bfloat16)
    mask_f = null_mask.reshape(_N, 1).astype(jnp.float32)
    b2 = head_b.reshape(1, _K)
    return _fused_loss(x, mask_f, o, centroids, head_w, b2)


# speedup vs baseline: 1.2885x; 1.2885x over previous
import functools

import jax
import jax.numpy as jnp
from jax.experimental import pallas as pl
from jax.experimental.pallas import tpu as pltpu

_B, _S = 128, 64
_D = 64
_P = 512
_K = 8192
_N = _B * _S
_EPS = 0.1

_R = 512
_KT = 1024


def _bound_body(o_ref, w_ref, b_ref, m_out_ref):
    ones = jnp.ones((_P, 1), jnp.bfloat16)
    o = o_ref[...]
    o2 = jax.lax.dot_general(o * o, ones, (((1,), (0,)), ((), ())),
                             preferred_element_type=jnp.float32)
    w = w_ref[...]
    w2 = jax.lax.dot_general(w * w, ones, (((1,), (0,)), ((), ())),
                             preferred_element_type=jnp.float32)
    omax = jnp.max(o2, keepdims=True)
    wmax = jnp.max(w2, keepdims=True)
    bmax = jnp.max(jnp.abs(b_ref[...]), keepdims=True)
    m_out_ref[...] = jnp.sqrt(omax * wmax) * 1.05 + bmax + 0.05


def _fused_body(m_in_ref, x_ref, c_ref, o_ref, w_ref, b_ref, mask_ref,
                out_ref, accn_ref, accd_ref, wsum_ref, bsum_ref, ca_ref):
    i = pl.program_id(0)
    ni = pl.num_programs(0)

    @pl.when(i == 0)
    def _init():
        accn_ref[...] = jnp.zeros_like(accn_ref)
        accd_ref[...] = jnp.zeros_like(accd_ref)
        wsum_ref[...] = jnp.sum(w_ref[...].astype(jnp.float32), axis=0,
                                keepdims=True)
        bsum_ref[...] = jnp.sum(b_ref[...], keepdims=True)
        cf = c_ref[...]
        c2f = jnp.sum(cf * cf, axis=0, keepdims=True)
        ca_ref[...] = jnp.concatenate(
            [cf.astype(jnp.bfloat16), (-0.5 * c2f).astype(jnp.bfloat16)],
            axis=0)

    mb = m_in_ref[0, 0].astype(jnp.bfloat16)
    xa = x_ref[...]
    o = o_ref[...]

    ones = jnp.ones((_KT, 1), jnp.bfloat16)
    _rowsum = lambda t: jax.lax.dot_general(
        t, ones, (((1,), (0,)), ((), ())), preferred_element_type=jnp.float32)

    esum = jnp.zeros((_R, _KT), jnp.bfloat16)
    best = jnp.full((_R, 1), -jnp.inf, jnp.float32)
    bp = jnp.zeros((_R, 1), jnp.float32)
    for jj in range(_K // _KT):
        ksl = slice(jj * _KT, (jj + 1) * _KT)
        ca = ca_ref[:, ksl]
        score = jnp.dot(xa, ca, preferred_element_type=jnp.float32)

        w = w_ref[ksl, :]
        pred32 = jax.lax.dot_general(o, w, (((1,), (1,)), ((), ())),
                                     preferred_element_type=jnp.float32)
        pred = (pred32 + b_ref[:, ksl]).astype(jnp.bfloat16)

        e = jnp.exp(pred - mb)
        esum = esum + e

        tb = jnp.max(score, axis=1, keepdims=True)
        pa = _rowsum(jnp.where(score == tb, pred, 0.0))
        upd = tb > best
        bp = jnp.where(upd, pa, bp)
        best = jnp.maximum(best, tb)

    s = _rowsum(esum)
    lse = mb.astype(jnp.float32) + jnp.log(s)
    nll = lse - bp
    ow = o.astype(jnp.float32) * wsum_ref[...]
    ps = jax.lax.dot_general(
        ow, jnp.ones((_P, 1), jnp.float32), (((1,), (0,)), ((), ())),
        preferred_element_type=jnp.float32) + bsum_ref[...]
    smooth = lse - ps * (1.0 / _K)
    elem = (1.0 - _EPS) * nll + _EPS * smooth
    mk = mask_ref[...]
    accn_ref[...] = accn_ref[...] + jnp.sum(mk * elem, keepdims=True)
    accd_ref[...] = accd_ref[...] + jnp.sum(mk, keepdims=True)

    @pl.when(i == ni - 1)
    def _emit():
        out_ref[...] = accn_ref[...] / accd_ref[...]


@functools.partial(jax.jit, static_argnames=("interpret",))
def _fused_loss(x, mask_f, o, centroids, head_w, head_b, interpret=False):
    m_bound = pl.pallas_call(
        _bound_body,
        out_shape=jax.ShapeDtypeStruct((1, 1), jnp.float32),
        interpret=interpret,
    )(o, head_w, head_b)

    grid = (_N // _R,)
    out = pl.pallas_call(
        _fused_body,
        grid=grid,
        in_specs=[
            pl.BlockSpec((1, 1), lambda i: (0, 0)),
            pl.BlockSpec((_R, _D + 1), lambda i: (i, 0)),
            pl.BlockSpec((_D, _K), lambda i: (0, 0)),
            pl.BlockSpec((_R, _P), lambda i: (i, 0)),
            pl.BlockSpec((_K, _P), lambda i: (0, 0)),
            pl.BlockSpec((1, _K), lambda i: (0, 0)),
            pl.BlockSpec((_R, 1), lambda i: (i, 0)),
        ],
        out_specs=pl.BlockSpec((1, 1), lambda i: (0, 0)),
        out_shape=jax.ShapeDtypeStruct((1, 1), jnp.float32),
        scratch_shapes=[
            pltpu.VMEM((1, 1), jnp.float32),
            pltpu.VMEM((1, 1), jnp.float32),
            pltpu.VMEM((1, _P), jnp.float32),
            pltpu.VMEM((1, 1), jnp.float32),
            pltpu.VMEM((_D + 1, _K), jnp.bfloat16),
        ],
        compiler_params=pltpu.CompilerParams(
            dimension_semantics=("arbitrary",),
        ),
        interpret=interpret,
    )(m_bound, x, centroids, o, head_w, head_b, mask_f)
    return out[0, 0]


def kernel(csts, null_mask, outputs, centroids, head_w, head_b):
    x = jnp.concatenate(
        [csts.reshape(_N, _D), jnp.ones((_N, 1), jnp.float32)],
        axis=1).astype(jnp.bfloat16)
    o = outputs.reshape(_N, _P).astype(jnp.bfloat16)
    head_w = head_w.astype(jnp.bfloat16)
    mask_f = null_mask.reshape(_N, 1).astype(jnp.float32)
    b2 = head_b.reshape(1, _K)
    return _fused_loss(x, mask_f, o, centroids, head_w, b2)


# speedup vs baseline: 1.2922x; 1.0029x over previous
import functools

import jax
import jax.numpy as jnp
from jax.experimental import pallas as pl
from jax.experimental.pallas import tpu as pltpu

_B, _S = 128, 64
_D = 64
_P = 512
_K = 8192
_N = _B * _S
_EPS = 0.1

_R = 1024
_KT = 2048


def _bound_body(o_ref, w_ref, b_ref, m_out_ref):
    ones = jnp.ones((_P, 1), jnp.bfloat16)
    o = o_ref[...]
    o2 = jax.lax.dot_general(o * o, ones, (((1,), (0,)), ((), ())),
                             preferred_element_type=jnp.float32)
    w = w_ref[...]
    w2 = jax.lax.dot_general(w * w, ones, (((1,), (0,)), ((), ())),
                             preferred_element_type=jnp.float32)
    omax = jnp.max(o2, keepdims=True)
    wmax = jnp.max(w2, keepdims=True)
    bmax = jnp.max(jnp.abs(b_ref[...]), keepdims=True)
    m_out_ref[...] = jnp.sqrt(omax * wmax) * 1.05 + bmax + 0.05


def _fused_body(m_in_ref, x_ref, c_ref, o_ref, w_ref, b_ref, mask_ref,
                out_ref, accn_ref, accd_ref, wsum_ref, bsum_ref, ca_ref):
    i = pl.program_id(0)
    ni = pl.num_programs(0)

    @pl.when(i == 0)
    def _init():
        accn_ref[...] = jnp.zeros_like(accn_ref)
        accd_ref[...] = jnp.zeros_like(accd_ref)
        wsum_ref[...] = jnp.sum(w_ref[...].astype(jnp.float32), axis=0,
                                keepdims=True)
        bsum_ref[...] = jnp.sum(b_ref[...], keepdims=True)
        cf = c_ref[...]
        c2f = jnp.sum(cf * cf, axis=0, keepdims=True)
        ca_ref[...] = jnp.concatenate(
            [cf.astype(jnp.bfloat16), (-0.5 * c2f).astype(jnp.bfloat16)],
            axis=0)

    mb = m_in_ref[0, 0].astype(jnp.bfloat16)
    xa = x_ref[...]
    o = o_ref[...]

    ones = jnp.ones((_KT, 1), jnp.bfloat16)
    _rowsum = lambda t: jax.lax.dot_general(
        t, ones, (((1,), (0,)), ((), ())), preferred_element_type=jnp.float32)

    esum = jnp.zeros((_R, _KT), jnp.bfloat16)
    best = jnp.full((_R, 1), -jnp.inf, jnp.float32)
    bp = jnp.zeros((_R, 1), jnp.float32)
    for jj in range(_K // _KT):
        ksl = slice(jj * _KT, (jj + 1) * _KT)
        ca = ca_ref[:, ksl]
        score = jnp.dot(xa, ca, preferred_element_type=jnp.float32)

        w = w_ref[ksl, :]
        pred32 = jax.lax.dot_general(o, w, (((1,), (1,)), ((), ())),
                                     preferred_element_type=jnp.float32)
        pred = (pred32 + b_ref[:, ksl]).astype(jnp.bfloat16)

        e = jnp.exp(pred - mb)
        esum = esum + e

        tb = jnp.max(score, axis=1, keepdims=True)
        pa = _rowsum(jnp.where(score == tb, pred, 0.0))
        upd = tb > best
        bp = jnp.where(upd, pa, bp)
        best = jnp.maximum(best, tb)

    s = _rowsum(esum)
    lse = mb.astype(jnp.float32) + jnp.log(s)
    nll = lse - bp
    ow = o.astype(jnp.float32) * wsum_ref[...]
    ps = jax.lax.dot_general(
        ow, jnp.ones((_P, 1), jnp.float32), (((1,), (0,)), ((), ())),
        preferred_element_type=jnp.float32) + bsum_ref[...]
    smooth = lse - ps * (1.0 / _K)
    elem = (1.0 - _EPS) * nll + _EPS * smooth
    mk = mask_ref[...]
    accn_ref[...] = accn_ref[...] + jnp.sum(mk * elem, keepdims=True)
    accd_ref[...] = accd_ref[...] + jnp.sum(mk, keepdims=True)

    @pl.when(i == ni - 1)
    def _emit():
        out_ref[...] = accn_ref[...] / accd_ref[...]


@functools.partial(jax.jit, static_argnames=("interpret",))
def _fused_loss(x, mask_f, o, centroids, head_w, head_b, interpret=False):
    m_bound = pl.pallas_call(
        _bound_body,
        out_shape=jax.ShapeDtypeStruct((1, 1), jnp.float32),
        interpret=interpret,
    )(o, head_w, head_b)

    grid = (_N // _R,)
    out = pl.pallas_call(
        _fused_body,
        grid=grid,
        in_specs=[
            pl.BlockSpec((1, 1), lambda i: (0, 0)),
            pl.BlockSpec((_R, _D + 1), lambda i: (i, 0)),
            pl.BlockSpec((_D, _K), lambda i: (0, 0)),
            pl.BlockSpec((_R, _P), lambda i: (i, 0)),
            pl.BlockSpec((_K, _P), lambda i: (0, 0)),
            pl.BlockSpec((1, _K), lambda i: (0, 0)),
            pl.BlockSpec((_R, 1), lambda i: (i, 0)),
        ],
        out_specs=pl.BlockSpec((1, 1), lambda i: (0, 0)),
        out_shape=jax.ShapeDtypeStruct((1, 1), jnp.float32),
        scratch_shapes=[
            pltpu.VMEM((1, 1), jnp.float32),
            pltpu.VMEM((1, 1), jnp.float32),
            pltpu.VMEM((1, _P), jnp.float32),
            pltpu.VMEM((1, 1), jnp.float32),
            pltpu.VMEM((_D + 1, _K), jnp.bfloat16),
        ],
        compiler_params=pltpu.CompilerParams(
            dimension_semantics=("arbitrary",),
        ),
        interpret=interpret,
    )(m_bound, x, centroids, o, head_w, head_b, mask_f)
    return out[0, 0]


def kernel(csts, null_mask, outputs, centroids, head_w, head_b):
    x = jnp.concatenate(
        [csts.reshape(_N, _D), jnp.ones((_N, 1), jnp.float32)],
        axis=1).astype(jnp.bfloat16)
    o = outputs.reshape(_N, _P).astype(jnp.bfloat16)
    head_w = head_w.astype(jnp.bfloat16)
    mask_f = null_mask.reshape(_N, 1).astype(jnp.float32)
    b2 = head_b.reshape(1, _K)
    return _fused_loss(x, mask_f, o, centroids, head_w, b2)


# speedup vs baseline: 1.3253x; 1.0257x over previous
import functools

import jax
import jax.numpy as jnp
from jax.experimental import pallas as pl
from jax.experimental.pallas import tpu as pltpu

_B, _S = 128, 64
_D = 64
_P = 512
_K = 8192
_N = _B * _S
_EPS = 0.1

_R = 1024
_KT = 1024


def _bound_body(o_ref, w_ref, b_ref, m_out_ref):
    ones = jnp.ones((_P, 1), jnp.bfloat16)
    o = o_ref[...]
    o2 = jax.lax.dot_general(o * o, ones, (((1,), (0,)), ((), ())),
                             preferred_element_type=jnp.float32)
    w = w_ref[...]
    w2 = jax.lax.dot_general(w * w, ones, (((1,), (0,)), ((), ())),
                             preferred_element_type=jnp.float32)
    omax = jnp.max(o2, keepdims=True)
    wmax = jnp.max(w2, keepdims=True)
    bmax = jnp.max(jnp.abs(b_ref[...]), keepdims=True)
    m_out_ref[...] = jnp.sqrt(omax * wmax) * 1.05 + bmax + 0.05


def _fused_body(m_in_ref, x_ref, c_ref, o_ref, w_ref, b_ref, mask_ref,
                out_ref, accn_ref, accd_ref, wsum_ref, bsum_ref, ca_ref,
                mshift_ref):
    i = pl.program_id(0)
    ni = pl.num_programs(0)

    @pl.when(i == 0)
    def _init():
        accn_ref[...] = jnp.zeros_like(accn_ref)
        accd_ref[...] = jnp.zeros_like(accd_ref)
        wsum_ref[...] = jnp.sum(w_ref[...].astype(jnp.float32), axis=0,
                                keepdims=True)
        bsum_ref[...] = jnp.sum(b_ref[...], keepdims=True)
        mshift_ref[...] = (m_in_ref[0, 0] - b_ref[...]).astype(jnp.bfloat16)
        cf = c_ref[...]
        c2f = jnp.sum(cf * cf, axis=0, keepdims=True)
        ca_ref[...] = jnp.concatenate(
            [cf.astype(jnp.bfloat16), (-0.5 * c2f).astype(jnp.bfloat16)],
            axis=0)

    mb = m_in_ref[0, 0].astype(jnp.bfloat16)
    xa = x_ref[...]
    o = o_ref[...]

    ones = jnp.ones((_KT, 1), jnp.bfloat16)
    _rowsum = lambda t: jax.lax.dot_general(
        t, ones, (((1,), (0,)), ((), ())), preferred_element_type=jnp.float32)

    esum = jnp.zeros((_R, _KT), jnp.bfloat16)
    best = jnp.full((_R, 1), -jnp.inf, jnp.float32)
    bp = jnp.zeros((_R, 1), jnp.float32)
    for jj in range(_K // _KT):
        ksl = slice(jj * _KT, (jj + 1) * _KT)
        ca = ca_ref[:, ksl]
        score = jnp.dot(xa, ca, preferred_element_type=jnp.float32)

        w = w_ref[ksl, :]
        pred32 = jax.lax.dot_general(o, w, (((1,), (1,)), ((), ())),
                                     preferred_element_type=jnp.float32)
        e = jnp.exp(pred32.astype(jnp.bfloat16) - mshift_ref[:, ksl])
        esum = esum + e

        tb = jnp.max(score, axis=1, keepdims=True)
        pa = _rowsum(jnp.where(score == tb, e, 0.0))
        upd = tb > best
        bp = jnp.where(upd, pa, bp)
        best = jnp.maximum(best, tb)

    s = _rowsum(esum)
    lse = mb.astype(jnp.float32) + jnp.log(s)
    nll = jnp.log(s) - jnp.log(bp)
    ow = o.astype(jnp.float32) * wsum_ref[...]
    ps = jax.lax.dot_general(
        ow, jnp.ones((_P, 1), jnp.float32), (((1,), (0,)), ((), ())),
        preferred_element_type=jnp.float32) + bsum_ref[...]
    smooth = lse - ps * (1.0 / _K)
    elem = (1.0 - _EPS) * nll + _EPS * smooth
    mk = mask_ref[...]
    accn_ref[...] = accn_ref[...] + jnp.sum(mk * elem, keepdims=True)
    accd_ref[...] = accd_ref[...] + jnp.sum(mk, keepdims=True)

    @pl.when(i == ni - 1)
    def _emit():
        out_ref[...] = accn_ref[...] / accd_ref[...]


@functools.partial(jax.jit, static_argnames=("interpret",))
def _fused_loss(x, mask_f, o, centroids, head_w, head_b, interpret=False):
    m_bound = pl.pallas_call(
        _bound_body,
        out_shape=jax.ShapeDtypeStruct((1, 1), jnp.float32),
        interpret=interpret,
    )(o, head_w, head_b)

    grid = (_N // _R,)
    out = pl.pallas_call(
        _fused_body,
        grid=grid,
        in_specs=[
            pl.BlockSpec((1, 1), lambda i: (0, 0)),
            pl.BlockSpec((_R, _D + 1), lambda i: (i, 0)),
            pl.BlockSpec((_D, _K), lambda i: (0, 0)),
            pl.BlockSpec((_R, _P), lambda i: (i, 0)),
            pl.BlockSpec((_K, _P), lambda i: (0, 0)),
            pl.BlockSpec((1, _K), lambda i: (0, 0)),
            pl.BlockSpec((_R, 1), lambda i: (i, 0)),
        ],
        out_specs=pl.BlockSpec((1, 1), lambda i: (0, 0)),
        out_shape=jax.ShapeDtypeStruct((1, 1), jnp.float32),
        scratch_shapes=[
            pltpu.VMEM((1, 1), jnp.float32),
            pltpu.VMEM((1, 1), jnp.float32),
            pltpu.VMEM((1, _P), jnp.float32),
            pltpu.VMEM((1, 1), jnp.float32),
            pltpu.VMEM((_D + 1, _K), jnp.bfloat16),
            pltpu.VMEM((1, _K), jnp.bfloat16),
        ],
        compiler_params=pltpu.CompilerParams(
            dimension_semantics=("arbitrary",),
        ),
        interpret=interpret,
    )(m_bound, x, centroids, o, head_w, head_b, mask_f)
    return out[0, 0]


def kernel(csts, null_mask, outputs, centroids, head_w, head_b):
    x = jnp.concatenate(
        [csts.reshape(_N, _D), jnp.ones((_N, 1), jnp.float32)],
        axis=1).astype(jnp.bfloat16)
    o = outputs.reshape(_N, _P).astype(jnp.bfloat16)
    head_w = head_w.astype(jnp.bfloat16)
    mask_f = null_mask.reshape(_N, 1).astype(jnp.float32)
    b2 = head_b.reshape(1, _K)
    return _fused_loss(x, mask_f, o, centroids, head_w, b2)
